# Initial kernel scaffold; baseline (speedup 1.0000x reference)
import jax
import jax.numpy as jnp
from jax.experimental import pallas as pl


def kernel(x, edge_index, W1, b1, W2, b2):
    raise NotImplementedError("write your pallas kernel here")


# speedup vs baseline: 27.8145x; 27.8145x over previous
import functools

import jax
import jax.numpy as jnp
from jax import lax
from jax.experimental import pallas as pl
from jax.experimental.pallas import tpu as pltpu
from jax.experimental.pallas import tpu_sc as plsc

N_NODES = 10000
NPAD = 10240
D_FEAT = 128
D_HID = 16
NC, NS = 2, 16
NW = NC * NS
CK = 128
CH = 80
EPAD = NW * CH * CK
RB = 1024
GRID = NPAD // RB

_mesh = plsc.VectorSubcoreMesh(core_axis_name="c", subcore_axis_name="s")


def _make_edge_scatter(d):
    tshape = (NPAD, d) if d else (NPAD,)
    rshape = (CK, d) if d else (CK,)
    rps = NPAD // NS

    @functools.partial(
        pl.kernel,
        mesh=_mesh,
        compiler_params=pltpu.CompilerParams(use_tc_tiling_on_sc=False),
        out_type=jax.ShapeDtypeStruct((NC,) + tshape, jnp.float32),
        scratch_types=[
            pltpu.VMEM((CH, CK), jnp.int32),
            pltpu.VMEM((CH, CK), jnp.int32),
            pltpu.VMEM(rshape, jnp.float32),
            pltpu.VMEM_SHARED(tshape, jnp.float32),
            pltpu.SemaphoreType.DMA,
        ],
    )
    def scat(table_hbm, src_hbm, dst_hbm, zeros_hbm, out_hbm,
             sidx, didx, rows, acc, sem):
        cid = lax.axis_index("c")
        sid = lax.axis_index("s")
        w = cid * NS + sid
        my = pl.ds(sid * rps, rps)
        pltpu.sync_copy(zeros_hbm.at[my], acc.at[my])
        pltpu.sync_copy(src_hbm.at[pl.ds(w * CH, CH)], sidx)
        pltpu.sync_copy(dst_hbm.at[pl.ds(w * CH, CH)], didx)
        plsc.subcore_barrier()

        def body(j, carry):
            pltpu.async_copy(table_hbm.at[sidx.at[j]], rows, sem).wait()
            pltpu.sync_copy(rows, acc.at[didx.at[j]], add=True)
            return carry

        lax.fori_loop(0, CH, body, 0)
        plsc.subcore_barrier()
        pltpu.sync_copy(acc.at[my], out_hbm.at[cid].at[my])

    return scat


_scat_rows = _make_edge_scatter(D_HID)
_scat_scalar = _make_edge_scatter(None)


def _s1_body(x_ref, w1_ref, degp_ref, g1_ref, dis_ref):
    deg = degp_ref[0, :] + degp_ref[1, :] + 1.0
    dis = lax.rsqrt(deg)
    h = jnp.dot(x_ref[...], w1_ref[...], preferred_element_type=jnp.float32)
    g1_ref[...] = h * dis[:, None]
    dis_ref[...] = dis[:, None]


def _s2_body(accp_ref, g1_ref, dis_ref, b1_ref, w2_ref, g2_ref):
    s = accp_ref[0] + accp_ref[1] + g1_ref[...]
    z = jnp.maximum(dis_ref[...] * s + b1_ref[...], 0.0)
    h2 = jnp.dot(z, w2_ref[...], preferred_element_type=jnp.float32)
    g2_ref[...] = dis_ref[...] * h2


def _s3_body(accp_ref, g2_ref, dis_ref, b2_ref, out_ref):
    s = accp_ref[0] + accp_ref[1]
    out_ref[...] = dis_ref[...] * (s[:, None] + g2_ref[...]) + b2_ref[...]


def kernel(x, edge_index, W1, b1, W2, b2):
    n, f = x.shape
    e = edge_index.shape[1]
    src = edge_index[0].astype(jnp.int32)
    dst = edge_index[1].astype(jnp.int32)
    src_p = jnp.concatenate(
        [src, jnp.zeros((EPAD - e,), jnp.int32)]).reshape(NW * CH, CK)
    dst_p = jnp.concatenate(
        [dst, jnp.full((EPAD - e,), n, jnp.int32)]).reshape(NW * CH, CK)
    xp = jnp.pad(x, ((0, NPAD - n), (0, 0)))
    zeros2 = jnp.zeros((NPAD, D_HID), jnp.float32)
    zeros1 = jnp.zeros((NPAD,), jnp.float32)
    ones1 = jnp.ones((NPAD,), jnp.float32)

    deg_p = _scat_scalar(ones1, dst_p, dst_p, zeros1)

    g1, dis = pl.pallas_call(
        _s1_body,
        grid=(GRID,),
        in_specs=[
            pl.BlockSpec((RB, D_FEAT), lambda i: (i, 0)),
            pl.BlockSpec((D_FEAT, D_HID), lambda i: (0, 0)),
            pl.BlockSpec((NC, RB), lambda i: (0, i)),
        ],
        out_specs=[
            pl.BlockSpec((RB, D_HID), lambda i: (i, 0)),
            pl.BlockSpec((RB, 1), lambda i: (i, 0)),
        ],
        out_shape=[
            jax.ShapeDtypeStruct((NPAD, D_HID), jnp.float32),
            jax.ShapeDtypeStruct((NPAD, 1), jnp.float32),
        ],
    )(xp, W1, deg_p)

    acc1_p = _scat_rows(g1, src_p, dst_p, zeros2)

    g2 = pl.pallas_call(
        _s2_body,
        grid=(GRID,),
        in_specs=[
            pl.BlockSpec((NC, RB, D_HID), lambda i: (0, i, 0)),
            pl.BlockSpec((RB, D_HID), lambda i: (i, 0)),
            pl.BlockSpec((RB, 1), lambda i: (i, 0)),
            pl.BlockSpec((1, D_HID), lambda i: (0, 0)),
            pl.BlockSpec((D_HID, 1), lambda i: (0, 0)),
        ],
        out_specs=pl.BlockSpec((RB, 1), lambda i: (i, 0)),
        out_shape=jax.ShapeDtypeStruct((NPAD, 1), jnp.float32),
    )(acc1_p, g1, dis, b1.reshape(1, D_HID), W2)

    acc2_p = _scat_scalar(g2.reshape(NPAD), src_p, dst_p, zeros1)

    out = pl.pallas_call(
        _s3_body,
        grid=(GRID,),
        in_specs=[
            pl.BlockSpec((NC, RB), lambda i: (0, i)),
            pl.BlockSpec((RB, 1), lambda i: (i, 0)),
            pl.BlockSpec((RB, 1), lambda i: (i, 0)),
            pl.BlockSpec((1, 1), lambda i: (0, 0)),
        ],
        out_specs=pl.BlockSpec((RB, 1), lambda i: (i, 0)),
        out_shape=jax.ShapeDtypeStruct((NPAD, 1), jnp.float32),
    )(acc2_p, g2, dis, b2.reshape(1, 1))

    return out[:n]


# speedup vs baseline: 39.2715x; 1.4119x over previous
import functools

import jax
import jax.numpy as jnp
from jax import lax
from jax.experimental import pallas as pl
from jax.experimental.pallas import tpu as pltpu
from jax.experimental.pallas import tpu_sc as plsc

N_NODES = 10000
NPAD = 10240
D_FEAT = 128
D_HID = 16
NC, NS = 2, 16
NW = NC * NS
CK = 128
CH = 80
EPAD = NW * CH * CK
RB = 1024
GRID = NPAD // RB

_mesh = plsc.VectorSubcoreMesh(core_axis_name="c", subcore_axis_name="s")


KBUF = 8


def _make_edge_scatter(d, gather=True):
    tshape = (NPAD, d) if d else (NPAD,)
    bshape = (KBUF, CK, d) if d else (KBUF, CK)
    rps = NPAD // NS

    @functools.partial(
        pl.kernel,
        mesh=_mesh,
        compiler_params=pltpu.CompilerParams(use_tc_tiling_on_sc=False),
        out_type=jax.ShapeDtypeStruct((NC,) + tshape, jnp.float32),
        scratch_types=[
            pltpu.VMEM((CH, CK), jnp.int32),
            pltpu.VMEM((CH, CK), jnp.int32),
            pltpu.VMEM(bshape, jnp.float32),
            pltpu.VMEM_SHARED(tshape, jnp.float32),
            pltpu.SemaphoreType.DMA,
            pltpu.SemaphoreType.DMA,
        ],
    )
    def scat(table_hbm, src_hbm, dst_hbm, zeros_hbm, out_hbm,
             sidx, didx, rows, acc, gsem, ssem):
        cid = lax.axis_index("c")
        sid = lax.axis_index("s")
        w = cid * NS + sid
        my = pl.ds(sid * rps, rps)
        pltpu.sync_copy(zeros_hbm.at[my], acc.at[my])
        pltpu.sync_copy(src_hbm.at[pl.ds(w * CH, CH)], sidx)
        pltpu.sync_copy(dst_hbm.at[pl.ds(w * CH, CH)], didx)
        if not gather:
            for b in range(KBUF):
                pltpu.sync_copy(table_hbm.at[pl.ds(0, CK)], rows.at[b])
        plsc.subcore_barrier()

        def body(i, carry):
            j = i * KBUF
            if gather:
                hs = [pltpu.async_copy(table_hbm.at[sidx.at[j + b]],
                                       rows.at[b], gsem)
                      for b in range(KBUF)]
                for h in hs:
                    h.wait()
            hs = [pltpu.async_copy(rows.at[b], acc.at[didx.at[j + b]],
                                   ssem, add=True)
                  for b in range(KBUF)]
            for h in hs:
                h.wait()
            return carry

        lax.fori_loop(0, CH // KBUF, body, 0)
        plsc.subcore_barrier()
        pltpu.sync_copy(acc.at[my], out_hbm.at[cid].at[my])

    return scat


_scat_rows = _make_edge_scatter(D_HID)
_scat_scalar = _make_edge_scatter(None)
_scat_const = _make_edge_scatter(None, gather=False)


def _s1_body(x_ref, w1_ref, degp_ref, g1_ref, dis_ref):
    deg = degp_ref[0, :] + degp_ref[1, :] + 1.0
    dis = lax.rsqrt(deg)
    h = jnp.dot(x_ref[...], w1_ref[...], preferred_element_type=jnp.float32)
    g1_ref[...] = h * dis[:, None]
    dis_ref[...] = dis[:, None]


def _s2_body(accp_ref, g1_ref, dis_ref, b1_ref, w2_ref, g2_ref):
    s = accp_ref[0] + accp_ref[1] + g1_ref[...]
    z = jnp.maximum(dis_ref[...] * s + b1_ref[...], 0.0)
    h2 = jnp.dot(z, w2_ref[...], preferred_element_type=jnp.float32)
    g2_ref[...] = dis_ref[...] * h2


def _s3_body(accp_ref, g2_ref, dis_ref, b2_ref, out_ref):
    s = accp_ref[0] + accp_ref[1]
    out_ref[...] = dis_ref[...] * (s[:, None] + g2_ref[...]) + b2_ref[...]


def kernel(x, edge_index, W1, b1, W2, b2):
    n, f = x.shape
    e = edge_index.shape[1]
    src = edge_index[0].astype(jnp.int32)
    dst = edge_index[1].astype(jnp.int32)
    src_p = jnp.concatenate(
        [src, jnp.zeros((EPAD - e,), jnp.int32)]).reshape(NW * CH, CK)
    dst_p = jnp.concatenate(
        [dst, jnp.full((EPAD - e,), n, jnp.int32)]).reshape(NW * CH, CK)
    xp = jnp.pad(x, ((0, NPAD - n), (0, 0)))
    zeros2 = jnp.zeros((NPAD, D_HID), jnp.float32)
    zeros1 = jnp.zeros((NPAD,), jnp.float32)
    ones1 = jnp.ones((NPAD,), jnp.float32)

    deg_p = _scat_const(ones1, dst_p, dst_p, zeros1)

    g1, dis = pl.pallas_call(
        _s1_body,
        grid=(GRID,),
        in_specs=[
            pl.BlockSpec((RB, D_FEAT), lambda i: (i, 0)),
            pl.BlockSpec((D_FEAT, D_HID), lambda i: (0, 0)),
            pl.BlockSpec((NC, RB), lambda i: (0, i)),
        ],
        out_specs=[
            pl.BlockSpec((RB, D_HID), lambda i: (i, 0)),
            pl.BlockSpec((RB, 1), lambda i: (i, 0)),
        ],
        out_shape=[
            jax.ShapeDtypeStruct((NPAD, D_HID), jnp.float32),
            jax.ShapeDtypeStruct((NPAD, 1), jnp.float32),
        ],
    )(xp, W1, deg_p)

    acc1_p = _scat_rows(g1, src_p, dst_p, zeros2)

    g2 = pl.pallas_call(
        _s2_body,
        grid=(GRID,),
        in_specs=[
            pl.BlockSpec((NC, RB, D_HID), lambda i: (0, i, 0)),
            pl.BlockSpec((RB, D_HID), lambda i: (i, 0)),
            pl.BlockSpec((RB, 1), lambda i: (i, 0)),
            pl.BlockSpec((1, D_HID), lambda i: (0, 0)),
            pl.BlockSpec((D_HID, 1), lambda i: (0, 0)),
        ],
        out_specs=pl.BlockSpec((RB, 1), lambda i: (i, 0)),
        out_shape=jax.ShapeDtypeStruct((NPAD, 1), jnp.float32),
    )(acc1_p, g1, dis, b1.reshape(1, D_HID), W2)

    acc2_p = _scat_scalar(g2.reshape(NPAD), src_p, dst_p, zeros1)

    out = pl.pallas_call(
        _s3_body,
        grid=(GRID,),
        in_specs=[
            pl.BlockSpec((NC, RB), lambda i: (0, i)),
            pl.BlockSpec((RB, 1), lambda i: (i, 0)),
            pl.BlockSpec((RB, 1), lambda i: (i, 0)),
            pl.BlockSpec((1, 1), lambda i: (0, 0)),
        ],
        out_specs=pl.BlockSpec((RB, 1), lambda i: (i, 0)),
        out_shape=jax.ShapeDtypeStruct((NPAD, 1), jnp.float32),
    )(acc2_p, g2, dis, b2.reshape(1, 1))

    return out[:n]


# speedup vs baseline: 43.3889x; 1.1048x over previous
import functools

import jax
import jax.numpy as jnp
from jax import lax
from jax.experimental import pallas as pl
from jax.experimental.pallas import tpu as pltpu
from jax.experimental.pallas import tpu_sc as plsc

N_NODES = 10000
NPAD = 10240
D_FEAT = 128
D_HID = 16
NC, NS = 2, 16
NW = NC * NS
CK = 128
CH = 80
EPAD = NW * CH * CK
RB = 1024
GRID = NPAD // RB

_mesh = plsc.VectorSubcoreMesh(core_axis_name="c", subcore_axis_name="s")


def _make_edge_scatter(d, gather=True):
    G = 4 if d else 1
    GC = CH // G
    NB = 2 if G > 1 else 1
    EW = CH * CK
    tshape = (NPAD, d) if d else (NPAD,)
    bshape = (NB, GC * CK, d) if d else (NB, GC * CK)
    rps = NPAD // NS

    @functools.partial(
        pl.kernel,
        mesh=_mesh,
        compiler_params=pltpu.CompilerParams(use_tc_tiling_on_sc=False),
        out_type=jax.ShapeDtypeStruct((NC,) + tshape, jnp.float32),
        scratch_types=[
            pltpu.VMEM((EW,), jnp.int32),
            pltpu.VMEM((CH, CK), jnp.int32),
            pltpu.VMEM(bshape, jnp.float32),
            pltpu.VMEM_SHARED(tshape, jnp.float32),
            pltpu.SemaphoreType.DMA,
            pltpu.SemaphoreType.DMA,
        ],
    )
    def scat(table_hbm, src_hbm, dst_hbm, zeros_hbm, out_hbm,
             sidx, didx, rows, acc, gsem, ssem):
        cid = lax.axis_index("c")
        sid = lax.axis_index("s")
        w = cid * NS + sid
        my = pl.ds(sid * rps, rps)
        pltpu.sync_copy(zeros_hbm.at[my], acc.at[my])
        if gather:
            pltpu.sync_copy(src_hbm.at[pl.ds(w * EW, EW)], sidx)
        else:
            pltpu.sync_copy(table_hbm, rows.at[0])
        pltpu.sync_copy(dst_hbm.at[pl.ds(w * CH, CH)], didx)
        plsc.subcore_barrier()

        def fire_scatters(buf, g):
            return [pltpu.async_copy(
                        buf.at[pl.ds(j * CK, CK)],
                        acc.at[didx.at[g * GC + j]], ssem, add=True)
                    for j in range(GC)]

        if gather:
            hs = [None] * G
            hs[0] = pltpu.async_copy(
                table_hbm.at[sidx.at[pl.ds(0, GC * CK)]], rows.at[0], gsem)
            for g in range(G):
                if g + 1 < G:
                    hs[g + 1] = pltpu.async_copy(
                        table_hbm.at[sidx.at[pl.ds((g + 1) * GC * CK, GC * CK)]],
                        rows.at[(g + 1) % NB], gsem)
                hs[g].wait()
                for h in fire_scatters(rows.at[g % NB], g):
                    h.wait()
        else:
            for h in fire_scatters(rows.at[0], 0):
                h.wait()
        plsc.subcore_barrier()
        pltpu.sync_copy(acc.at[my], out_hbm.at[cid].at[my])

    return scat


_scat_rows = _make_edge_scatter(D_HID)
_scat_scalar = _make_edge_scatter(None)
_scat_const = _make_edge_scatter(None, gather=False)


def _s1_body(x_ref, w1_ref, degp_ref, g1_ref, dis_ref):
    deg = degp_ref[0, :] + degp_ref[1, :] + 1.0
    dis = lax.rsqrt(deg)
    h = jnp.dot(x_ref[...], w1_ref[...], preferred_element_type=jnp.float32)
    g1_ref[...] = h * dis[:, None]
    dis_ref[...] = dis[:, None]


def _s2_body(accp_ref, g1_ref, dis_ref, b1_ref, w2_ref, g2_ref):
    s = accp_ref[0] + accp_ref[1] + g1_ref[...]
    z = jnp.maximum(dis_ref[...] * s + b1_ref[...], 0.0)
    h2 = jnp.dot(z, w2_ref[...], preferred_element_type=jnp.float32)
    g2_ref[...] = dis_ref[...] * h2


def _s3_body(accp_ref, g2_ref, dis_ref, b2_ref, out_ref):
    s = accp_ref[0] + accp_ref[1]
    out_ref[...] = dis_ref[...] * (s[:, None] + g2_ref[...]) + b2_ref[...]


def kernel(x, edge_index, W1, b1, W2, b2):
    n, f = x.shape
    e = edge_index.shape[1]
    src = edge_index[0].astype(jnp.int32)
    dst = edge_index[1].astype(jnp.int32)
    src_p = jnp.concatenate([src, jnp.zeros((EPAD - e,), jnp.int32)])
    dst_p = jnp.concatenate(
        [dst, jnp.full((EPAD - e,), n, jnp.int32)]).reshape(NW * CH, CK)
    xp = jnp.pad(x, ((0, NPAD - n), (0, 0)))
    zeros2 = jnp.zeros((NPAD, D_HID), jnp.float32)
    zeros1 = jnp.zeros((NPAD,), jnp.float32)
    ones1 = jnp.ones((CH * CK,), jnp.float32)

    deg_p = _scat_const(ones1, src_p, dst_p, zeros1)

    g1, dis = pl.pallas_call(
        _s1_body,
        grid=(GRID,),
        in_specs=[
            pl.BlockSpec((RB, D_FEAT), lambda i: (i, 0)),
            pl.BlockSpec((D_FEAT, D_HID), lambda i: (0, 0)),
            pl.BlockSpec((NC, RB), lambda i: (0, i)),
        ],
        out_specs=[
            pl.BlockSpec((RB, D_HID), lambda i: (i, 0)),
            pl.BlockSpec((RB, 1), lambda i: (i, 0)),
        ],
        out_shape=[
            jax.ShapeDtypeStruct((NPAD, D_HID), jnp.float32),
            jax.ShapeDtypeStruct((NPAD, 1), jnp.float32),
        ],
    )(xp, W1, deg_p)

    acc1_p = _scat_rows(g1, src_p, dst_p, zeros2)

    g2 = pl.pallas_call(
        _s2_body,
        grid=(GRID,),
        in_specs=[
            pl.BlockSpec((NC, RB, D_HID), lambda i: (0, i, 0)),
            pl.BlockSpec((RB, D_HID), lambda i: (i, 0)),
            pl.BlockSpec((RB, 1), lambda i: (i, 0)),
            pl.BlockSpec((1, D_HID), lambda i: (0, 0)),
            pl.BlockSpec((D_HID, 1), lambda i: (0, 0)),
        ],
        out_specs=pl.BlockSpec((RB, 1), lambda i: (i, 0)),
        out_shape=jax.ShapeDtypeStruct((NPAD, 1), jnp.float32),
    )(acc1_p, g1, dis, b1.reshape(1, D_HID), W2)

    acc2_p = _scat_scalar(g2.reshape(NPAD), src_p, dst_p, zeros1)

    out = pl.pallas_call(
        _s3_body,
        grid=(GRID,),
        in_specs=[
            pl.BlockSpec((NC, RB), lambda i: (0, i)),
            pl.BlockSpec((RB, 1), lambda i: (i, 0)),
            pl.BlockSpec((RB, 1), lambda i: (i, 0)),
            pl.BlockSpec((1, 1), lambda i: (0, 0)),
        ],
        out_specs=pl.BlockSpec((RB, 1), lambda i: (i, 0)),
        out_shape=jax.ShapeDtypeStruct((NPAD, 1), jnp.float32),
    )(acc2_p, g2, dis, b2.reshape(1, 1))

    return out[:n]


# speedup vs baseline: 55.3035x; 1.2746x over previous
import functools

import jax
import jax.numpy as jnp
from jax import lax
from jax.experimental import pallas as pl
from jax.experimental.pallas import tpu as pltpu
from jax.experimental.pallas import tpu_sc as plsc

N_NODES = 10000
NPAD = 10240
D_FEAT = 128
D_HID = 16
NC, NS = 2, 16
NW = NC * NS
CK = 128
CH = 80
EPAD = NW * CH * CK
RB = 1024
GRID = NPAD // RB

_mesh = plsc.VectorSubcoreMesh(core_axis_name="c", subcore_axis_name="s")


def _make_edge_scatter(d, gather=True):
    G = 4 if d else 1
    GC = CH // G
    NB = 2 if G > 1 else 1
    EW = CH * CK
    tshape = (NPAD, d) if d else (NPAD,)
    bshape = (NB, GC * CK, d) if d else (NB, GC * CK)
    rps = NPAD // NS

    @functools.partial(
        pl.kernel,
        mesh=_mesh,
        compiler_params=pltpu.CompilerParams(use_tc_tiling_on_sc=False),
        out_type=jax.ShapeDtypeStruct((NC,) + tshape, jnp.float32),
        scratch_types=[
            pltpu.VMEM((EW,), jnp.int32),
            pltpu.VMEM((CH, CK), jnp.int32),
            pltpu.VMEM(bshape, jnp.float32),
            pltpu.VMEM_SHARED(tshape, jnp.float32),
            pltpu.SemaphoreType.DMA,
            pltpu.SemaphoreType.DMA,
        ],
    )
    def scat(table_hbm, src_hbm, dst_hbm, zeros_hbm, out_hbm,
             sidx, didx, rows, acc, gsem, ssem):
        cid = lax.axis_index("c")
        sid = lax.axis_index("s")
        w = cid * NS + sid
        my = pl.ds(sid * rps, rps)
        pltpu.sync_copy(zeros_hbm.at[my], acc.at[my])
        if gather:
            pltpu.sync_copy(src_hbm.at[pl.ds(w * EW, EW)], sidx)
        else:
            pltpu.sync_copy(table_hbm, rows.at[0])
        pltpu.sync_copy(dst_hbm.at[pl.ds(w * CH, CH)], didx)
        plsc.subcore_barrier()

        def fire_scatters(buf, g):
            return [pltpu.async_copy(
                        buf.at[pl.ds(j * CK, CK)],
                        acc.at[didx.at[g * GC + j]], ssem, add=True)
                    for j in range(GC)]

        if gather:
            hs = [None] * G
            hs[0] = pltpu.async_copy(
                table_hbm.at[sidx.at[pl.ds(0, GC * CK)]], rows.at[0], gsem)
            for g in range(G):
                if g + 1 < G:
                    hs[g + 1] = pltpu.async_copy(
                        table_hbm.at[sidx.at[pl.ds((g + 1) * GC * CK, GC * CK)]],
                        rows.at[(g + 1) % NB], gsem)
                hs[g].wait()
                for h in fire_scatters(rows.at[g % NB], g):
                    h.wait()
        else:
            for h in fire_scatters(rows.at[0], 0):
                h.wait()
        plsc.subcore_barrier()
        pltpu.sync_copy(acc.at[my], out_hbm.at[cid].at[my])

    return scat


_scat_rows = _make_edge_scatter(D_HID)
_scat_const = _make_edge_scatter(None, gather=False)

_EW = CH * CK
_GS = 4
_EG = _EW // _GS
_rps = NPAD // NS


@functools.partial(
    pl.kernel,
    mesh=_mesh,
    compiler_params=pltpu.CompilerParams(use_tc_tiling_on_sc=False,
                                         needs_layout_passes=False),
    out_type=jax.ShapeDtypeStruct((NC, NPAD), jnp.float32),
    scratch_types=[
        pltpu.VMEM((NPAD,), jnp.float32),
        pltpu.VMEM((_EW,), jnp.int32),
        pltpu.VMEM((CH, CK), jnp.int32),
        pltpu.VMEM((_EW,), jnp.float32),
        pltpu.VMEM_SHARED((NPAD,), jnp.float32),
        pltpu.SemaphoreType.DMA,
    ],
)
def _scat_scalar(table_hbm, src_hbm, dst_hbm, zeros_hbm, out_hbm,
                 tabv, sidx, didx, vals, acc, ssem):
    cid = lax.axis_index("c")
    sid = lax.axis_index("s")
    w = cid * NS + sid
    my = pl.ds(sid * _rps, _rps)
    pltpu.sync_copy(zeros_hbm.at[my], acc.at[my])
    pltpu.sync_copy(table_hbm, tabv)
    pltpu.sync_copy(src_hbm.at[pl.ds(w * _EW, _EW)], sidx)
    pltpu.sync_copy(dst_hbm.at[pl.ds(w * CH, CH)], didx)
    plsc.subcore_barrier()

    hs = []
    for g in range(_GS):
        def gbody(i, carry, g=g):
            k = g * _EG + i * 16
            idx = sidx[pl.ds(k, 16)]
            vals[pl.ds(k, 16)] = plsc.load_gather(tabv, [idx])
            return carry

        lax.fori_loop(0, _EG // 16, gbody, 0)
        hs += [pltpu.async_copy(
                   vals.at[pl.ds((g * (CH // _GS) + j) * CK, CK)],
                   acc.at[didx.at[g * (CH // _GS) + j]], ssem, add=True)
               for j in range(CH // _GS)]
    for h in hs:
        h.wait()
    plsc.subcore_barrier()
    pltpu.sync_copy(acc.at[my], out_hbm.at[cid].at[my])


def _s1_body(x_ref, w1_ref, degp_ref, g1_ref, dis_ref):
    deg = degp_ref[0, :] + degp_ref[1, :] + 1.0
    dis = lax.rsqrt(deg)
    h = jnp.dot(x_ref[...], w1_ref[...], preferred_element_type=jnp.float32)
    g1_ref[...] = h * dis[:, None]
    dis_ref[...] = dis[:, None]


def _s2_body(accp_ref, g1_ref, dis_ref, b1_ref, w2_ref, g2_ref):
    s = accp_ref[0] + accp_ref[1] + g1_ref[...]
    z = jnp.maximum(dis_ref[...] * s + b1_ref[...], 0.0)
    h2 = jnp.dot(z, w2_ref[...], preferred_element_type=jnp.float32)
    g2_ref[...] = dis_ref[...] * h2


def _s3_body(accp_ref, g2_ref, dis_ref, b2_ref, out_ref):
    s = accp_ref[0] + accp_ref[1]
    out_ref[...] = dis_ref[...] * (s[:, None] + g2_ref[...]) + b2_ref[...]


def kernel(x, edge_index, W1, b1, W2, b2):
    n, f = x.shape
    e = edge_index.shape[1]
    src = edge_index[0].astype(jnp.int32)
    dst = edge_index[1].astype(jnp.int32)
    src_p = jnp.concatenate([src, jnp.zeros((EPAD - e,), jnp.int32)])
    dst_p = jnp.concatenate(
        [dst, jnp.full((EPAD - e,), n, jnp.int32)]).reshape(NW * CH, CK)
    xp = jnp.pad(x, ((0, NPAD - n), (0, 0)))
    zeros2 = jnp.zeros((NPAD, D_HID), jnp.float32)
    zeros1 = jnp.zeros((NPAD,), jnp.float32)
    ones1 = jnp.ones((CH * CK,), jnp.float32)

    deg_p = _scat_const(ones1, src_p, dst_p, zeros1)

    g1, dis = pl.pallas_call(
        _s1_body,
        grid=(GRID,),
        in_specs=[
            pl.BlockSpec((RB, D_FEAT), lambda i: (i, 0)),
            pl.BlockSpec((D_FEAT, D_HID), lambda i: (0, 0)),
            pl.BlockSpec((NC, RB), lambda i: (0, i)),
        ],
        out_specs=[
            pl.BlockSpec((RB, D_HID), lambda i: (i, 0)),
            pl.BlockSpec((RB, 1), lambda i: (i, 0)),
        ],
        out_shape=[
            jax.ShapeDtypeStruct((NPAD, D_HID), jnp.float32),
            jax.ShapeDtypeStruct((NPAD, 1), jnp.float32),
        ],
    )(xp, W1, deg_p)

    acc1_p = _scat_rows(g1, src_p, dst_p, zeros2)

    g2 = pl.pallas_call(
        _s2_body,
        grid=(GRID,),
        in_specs=[
            pl.BlockSpec((NC, RB, D_HID), lambda i: (0, i, 0)),
            pl.BlockSpec((RB, D_HID), lambda i: (i, 0)),
            pl.BlockSpec((RB, 1), lambda i: (i, 0)),
            pl.BlockSpec((1, D_HID), lambda i: (0, 0)),
            pl.BlockSpec((D_HID, 1), lambda i: (0, 0)),
        ],
        out_specs=pl.BlockSpec((RB, 1), lambda i: (i, 0)),
        out_shape=jax.ShapeDtypeStruct((NPAD, 1), jnp.float32),
    )(acc1_p, g1, dis, b1.reshape(1, D_HID), W2)

    acc2_p = _scat_scalar(g2.reshape(NPAD), src_p, dst_p, zeros1)

    out = pl.pallas_call(
        _s3_body,
        grid=(GRID,),
        in_specs=[
            pl.BlockSpec((NC, RB), lambda i: (0, i)),
            pl.BlockSpec((RB, 1), lambda i: (i, 0)),
            pl.BlockSpec((RB, 1), lambda i: (i, 0)),
            pl.BlockSpec((1, 1), lambda i: (0, 0)),
        ],
        out_specs=pl.BlockSpec((RB, 1), lambda i: (i, 0)),
        out_shape=jax.ShapeDtypeStruct((NPAD, 1), jnp.float32),
    )(acc2_p, g2, dis, b2.reshape(1, 1))

    return out[:n]


# speedup vs baseline: 55.3561x; 1.0010x over previous
import functools

import jax
import jax.numpy as jnp
from jax import lax
from jax.experimental import pallas as pl
from jax.experimental.pallas import tpu as pltpu
from jax.experimental.pallas import tpu_sc as plsc

N_NODES = 10000
NPAD = 10240
D_FEAT = 128
D_HID = 16
NC, NS = 2, 16
NW = NC * NS
CK = 128
CH = 80
EPAD = NW * CH * CK
RB = 1024
GRID = NPAD // RB

_mesh = plsc.VectorSubcoreMesh(core_axis_name="c", subcore_axis_name="s")


def _make_edge_scatter(d, gather=True):
    G = 4 if d else 1
    GC = CH // G
    NB = 2 if G > 1 else 1
    EW = CH * CK
    tshape = (NPAD, d) if d else (NPAD,)
    bshape = (NB, GC * CK, d) if d else (NB, GC * CK)
    rps = NPAD // NS

    @functools.partial(
        pl.kernel,
        mesh=_mesh,
        compiler_params=pltpu.CompilerParams(use_tc_tiling_on_sc=False),
        out_type=jax.ShapeDtypeStruct((NC,) + tshape, jnp.float32),
        scratch_types=[
            pltpu.VMEM((EW,), jnp.int32),
            pltpu.VMEM((CH, CK), jnp.int32),
            pltpu.VMEM(bshape, jnp.float32),
            pltpu.VMEM_SHARED(tshape, jnp.float32),
            pltpu.SemaphoreType.DMA,
            pltpu.SemaphoreType.DMA,
        ],
    )
    def scat(table_hbm, src_hbm, dst_hbm, zeros_hbm, out_hbm,
             sidx, didx, rows, acc, gsem, ssem):
        cid = lax.axis_index("c")
        sid = lax.axis_index("s")
        w = cid * NS + sid
        my = pl.ds(sid * rps, rps)
        pltpu.sync_copy(zeros_hbm.at[my], acc.at[my])
        if gather:
            pltpu.sync_copy(src_hbm.at[pl.ds(w * EW, EW)], sidx)
        else:
            pltpu.sync_copy(table_hbm, rows.at[0])
        pltpu.sync_copy(dst_hbm.at[pl.ds(w * CH, CH)], didx)
        plsc.subcore_barrier()

        def fire_scatters(buf, g):
            return [pltpu.async_copy(
                        buf.at[pl.ds(j * CK, CK)],
                        acc.at[didx.at[g * GC + j]], ssem, add=True)
                    for j in range(GC)]

        if gather:
            hs = [None] * G
            ss = [None] * G
            hs[0] = pltpu.async_copy(
                table_hbm.at[sidx.at[pl.ds(0, GC * CK)]], rows.at[0], gsem)
            for g in range(G):
                if g + 1 < G:
                    if g >= 1:
                        for h in ss[g - 1]:
                            h.wait()
                    hs[g + 1] = pltpu.async_copy(
                        table_hbm.at[sidx.at[pl.ds((g + 1) * GC * CK, GC * CK)]],
                        rows.at[(g + 1) % NB], gsem)
                hs[g].wait()
                ss[g] = fire_scatters(rows.at[g % NB], g)
            for g in (G - 2, G - 1):
                for h in ss[g]:
                    h.wait()
        else:
            for h in fire_scatters(rows.at[0], 0):
                h.wait()
        plsc.subcore_barrier()
        pltpu.sync_copy(acc.at[my], out_hbm.at[cid].at[my])

    return scat


_scat_rows = _make_edge_scatter(D_HID)
_scat_const = _make_edge_scatter(None, gather=False)

_EW = CH * CK
_GS = 4
_EG = _EW // _GS
_rps = NPAD // NS


@functools.partial(
    pl.kernel,
    mesh=_mesh,
    compiler_params=pltpu.CompilerParams(use_tc_tiling_on_sc=False,
                                         needs_layout_passes=False),
    out_type=jax.ShapeDtypeStruct((NC, NPAD), jnp.float32),
    scratch_types=[
        pltpu.VMEM((NPAD,), jnp.float32),
        pltpu.VMEM((_EW,), jnp.int32),
        pltpu.VMEM((CH, CK), jnp.int32),
        pltpu.VMEM((_EW,), jnp.float32),
        pltpu.VMEM_SHARED((NPAD,), jnp.float32),
        pltpu.SemaphoreType.DMA,
    ],
)
def _scat_scalar(table_hbm, src_hbm, dst_hbm, zeros_hbm, out_hbm,
                 tabv, sidx, didx, vals, acc, ssem):
    cid = lax.axis_index("c")
    sid = lax.axis_index("s")
    w = cid * NS + sid
    my = pl.ds(sid * _rps, _rps)
    pltpu.sync_copy(zeros_hbm.at[my], acc.at[my])
    pltpu.sync_copy(table_hbm, tabv)
    pltpu.sync_copy(src_hbm.at[pl.ds(w * _EW, _EW)], sidx)
    pltpu.sync_copy(dst_hbm.at[pl.ds(w * CH, CH)], didx)
    plsc.subcore_barrier()

    hs = []
    for g in range(_GS):
        def gbody(i, carry, g=g):
            k = g * _EG + i * 16
            idx = sidx[pl.ds(k, 16)]
            vals[pl.ds(k, 16)] = plsc.load_gather(tabv, [idx])
            return carry

        lax.fori_loop(0, _EG // 16, gbody, 0)
        hs += [pltpu.async_copy(
                   vals.at[pl.ds((g * (CH // _GS) + j) * CK, CK)],
                   acc.at[didx.at[g * (CH // _GS) + j]], ssem, add=True)
               for j in range(CH // _GS)]
    for h in hs:
        h.wait()
    plsc.subcore_barrier()
    pltpu.sync_copy(acc.at[my], out_hbm.at[cid].at[my])


def _s1_body(x_ref, w1_ref, degp_ref, g1_ref, dis_ref):
    deg = degp_ref[0, :] + degp_ref[1, :] + 1.0
    dis = lax.rsqrt(deg)
    h = jnp.dot(x_ref[...], w1_ref[...], preferred_element_type=jnp.float32)
    g1_ref[...] = h * dis[:, None]
    dis_ref[...] = dis[:, None]


def _s2_body(accp_ref, g1_ref, dis_ref, b1_ref, w2_ref, g2_ref):
    s = accp_ref[0] + accp_ref[1] + g1_ref[...]
    z = jnp.maximum(dis_ref[...] * s + b1_ref[...], 0.0)
    h2 = jnp.dot(z, w2_ref[...], preferred_element_type=jnp.float32)
    g2_ref[...] = dis_ref[...] * h2


def _s3_body(accp_ref, g2_ref, dis_ref, b2_ref, out_ref):
    s = accp_ref[0] + accp_ref[1]
    out_ref[...] = dis_ref[...] * (s[:, None] + g2_ref[...]) + b2_ref[...]


def kernel(x, edge_index, W1, b1, W2, b2):
    n, f = x.shape
    e = edge_index.shape[1]
    src = edge_index[0].astype(jnp.int32)
    dst = edge_index[1].astype(jnp.int32)
    src_p = jnp.concatenate([src, jnp.zeros((EPAD - e,), jnp.int32)])
    dst_p = jnp.concatenate(
        [dst, jnp.full((EPAD - e,), n, jnp.int32)]).reshape(NW * CH, CK)
    xp = jnp.pad(x, ((0, NPAD - n), (0, 0)))
    zeros2 = jnp.zeros((NPAD, D_HID), jnp.float32)
    zeros1 = jnp.zeros((NPAD,), jnp.float32)
    ones1 = jnp.ones((CH * CK,), jnp.float32)

    deg_p = _scat_const(ones1, src_p, dst_p, zeros1)

    g1, dis = pl.pallas_call(
        _s1_body,
        grid=(GRID,),
        in_specs=[
            pl.BlockSpec((RB, D_FEAT), lambda i: (i, 0)),
            pl.BlockSpec((D_FEAT, D_HID), lambda i: (0, 0)),
            pl.BlockSpec((NC, RB), lambda i: (0, i)),
        ],
        out_specs=[
            pl.BlockSpec((RB, D_HID), lambda i: (i, 0)),
            pl.BlockSpec((RB, 1), lambda i: (i, 0)),
        ],
        out_shape=[
            jax.ShapeDtypeStruct((NPAD, D_HID), jnp.float32),
            jax.ShapeDtypeStruct((NPAD, 1), jnp.float32),
        ],
    )(xp, W1, deg_p)

    acc1_p = _scat_rows(g1, src_p, dst_p, zeros2)

    g2 = pl.pallas_call(
        _s2_body,
        grid=(GRID,),
        in_specs=[
            pl.BlockSpec((NC, RB, D_HID), lambda i: (0, i, 0)),
            pl.BlockSpec((RB, D_HID), lambda i: (i, 0)),
            pl.BlockSpec((RB, 1), lambda i: (i, 0)),
            pl.BlockSpec((1, D_HID), lambda i: (0, 0)),
            pl.BlockSpec((D_HID, 1), lambda i: (0, 0)),
        ],
        out_specs=pl.BlockSpec((RB, 1), lambda i: (i, 0)),
        out_shape=jax.ShapeDtypeStruct((NPAD, 1), jnp.float32),
    )(acc1_p, g1, dis, b1.reshape(1, D_HID), W2)

    acc2_p = _scat_scalar(g2.reshape(NPAD), src_p, dst_p, zeros1)

    out = pl.pallas_call(
        _s3_body,
        grid=(GRID,),
        in_specs=[
            pl.BlockSpec((NC, RB), lambda i: (0, i)),
            pl.BlockSpec((RB, 1), lambda i: (i, 0)),
            pl.BlockSpec((RB, 1), lambda i: (i, 0)),
            pl.BlockSpec((1, 1), lambda i: (0, 0)),
        ],
        out_specs=pl.BlockSpec((RB, 1), lambda i: (i, 0)),
        out_shape=jax.ShapeDtypeStruct((NPAD, 1), jnp.float32),
    )(acc2_p, g2, dis, b2.reshape(1, 1))

    return out[:n]


# speedup vs baseline: 56.0102x; 1.0118x over previous
import functools

import jax
import jax.numpy as jnp
from jax import lax
from jax.experimental import pallas as pl
from jax.experimental.pallas import tpu as pltpu
from jax.experimental.pallas import tpu_sc as plsc

N_NODES = 10000
NPAD = 10240
D_FEAT = 128
D_HID = 16
NC, NS = 2, 16
NW = NC * NS
CK = 128
CH = 80
EPAD = NW * CH * CK
RB = 1024
GRID = NPAD // RB

_mesh = plsc.VectorSubcoreMesh(core_axis_name="c", subcore_axis_name="s")


def _make_edge_scatter(d, gather=True):
    G = 4 if d else 1
    GC = CH // G
    NB = 2 if G > 1 else 1
    EW = CH * CK
    tshape = (NPAD, d) if d else (NPAD,)
    bshape = (NB, GC * CK, d) if d else (NB, GC * CK)
    rps = NPAD // NS

    @functools.partial(
        pl.kernel,
        mesh=_mesh,
        compiler_params=pltpu.CompilerParams(use_tc_tiling_on_sc=False),
        out_type=jax.ShapeDtypeStruct((NC,) + tshape, jnp.float32),
        scratch_types=[
            pltpu.VMEM((EW,), jnp.int32),
            pltpu.VMEM((CH, CK), jnp.int32),
            pltpu.VMEM(bshape, jnp.float32),
            pltpu.VMEM_SHARED(tshape, jnp.float32),
            pltpu.SemaphoreType.DMA,
            pltpu.SemaphoreType.DMA,
        ],
    )
    def scat(table_hbm, src_hbm, dst_hbm, zeros_hbm, out_hbm,
             sidx, didx, rows, acc, gsem, ssem):
        cid = lax.axis_index("c")
        sid = lax.axis_index("s")
        w = cid * NS + sid
        my = pl.ds(sid * rps, rps)
        pltpu.sync_copy(zeros_hbm.at[my], acc.at[my])
        if gather:
            pltpu.sync_copy(src_hbm.at[pl.ds(w * EW, EW)], sidx)
        else:
            pltpu.sync_copy(table_hbm, rows.at[0])
        pltpu.sync_copy(dst_hbm.at[pl.ds(w * CH, CH)], didx)
        plsc.subcore_barrier()

        def fire_scatters(buf, g):
            return [pltpu.async_copy(
                        buf.at[pl.ds(j * CK, CK)],
                        acc.at[didx.at[g * GC + j]], ssem, add=True)
                    for j in range(GC)]

        if gather:
            hs = [None] * G
            ss = [None] * G
            hs[0] = pltpu.async_copy(
                table_hbm.at[sidx.at[pl.ds(0, GC * CK)]], rows.at[0], gsem)
            for g in range(G):
                if g + 1 < G:
                    if g >= 1:
                        for h in ss[g - 1]:
                            h.wait()
                    hs[g + 1] = pltpu.async_copy(
                        table_hbm.at[sidx.at[pl.ds((g + 1) * GC * CK, GC * CK)]],
                        rows.at[(g + 1) % NB], gsem)
                hs[g].wait()
                ss[g] = fire_scatters(rows.at[g % NB], g)
            for g in (G - 2, G - 1):
                for h in ss[g]:
                    h.wait()
        else:
            for h in fire_scatters(rows.at[0], 0):
                h.wait()
        plsc.subcore_barrier()
        pltpu.sync_copy(acc.at[my], out_hbm.at[cid].at[my])

    return scat


_scat_rows = _make_edge_scatter(D_HID)
_scat_const = _make_edge_scatter(None, gather=False)

_EW = CH * CK
_GS = 4
_EG = _EW // _GS
_rps = NPAD // NS


@functools.partial(
    pl.kernel,
    mesh=_mesh,
    compiler_params=pltpu.CompilerParams(use_tc_tiling_on_sc=False,
                                         needs_layout_passes=False),
    out_type=jax.ShapeDtypeStruct((NC, NPAD), jnp.float32),
    scratch_types=[
        pltpu.VMEM((N_NODES,), jnp.float32),
        pltpu.VMEM((_EW,), jnp.int32),
        pltpu.VMEM((CH, CK), jnp.int32),
        pltpu.VMEM((_EW,), jnp.float32),
        pltpu.VMEM_SHARED((NPAD,), jnp.float32),
        pltpu.SemaphoreType.DMA,
    ],
)
def _scat_scalar(table_hbm, src_hbm, dst_hbm, zeros_hbm, out_hbm,
                 tabv, sidx, didx, vals, acc, ssem):
    cid = lax.axis_index("c")
    sid = lax.axis_index("s")
    w = cid * NS + sid
    my = pl.ds(sid * _rps, _rps)
    pltpu.sync_copy(zeros_hbm.at[my], acc.at[my])
    pltpu.sync_copy(table_hbm, tabv)
    pltpu.sync_copy(src_hbm.at[pl.ds(w * _EW, _EW)], sidx)
    pltpu.sync_copy(dst_hbm.at[pl.ds(w * CH, CH)], didx)
    plsc.subcore_barrier()

    hs = []
    for g in range(_GS):
        def gbody(i, carry, g=g):
            k = g * _EG + i * 16
            idx = sidx[pl.ds(k, 16)]
            vals[pl.ds(k, 16)] = plsc.load_gather(tabv, [idx])
            return carry

        lax.fori_loop(0, _EG // 16, gbody, 0)
        hs += [pltpu.async_copy(
                   vals.at[pl.ds((g * (CH // _GS) + j) * CK, CK)],
                   acc.at[didx.at[g * (CH // _GS) + j]], ssem, add=True)
               for j in range(CH // _GS)]
    for h in hs:
        h.wait()
    plsc.subcore_barrier()
    pltpu.sync_copy(acc.at[my], out_hbm.at[cid].at[my])


def _s1_body(x_ref, w1_ref, degp_ref, g1_ref, dis_ref):
    n = x_ref.shape[0]
    deg = degp_ref[0, :n] + degp_ref[1, :n] + 1.0
    dis = lax.rsqrt(deg)
    h = jnp.dot(x_ref[...], w1_ref[...], preferred_element_type=jnp.float32)
    g1_ref[...] = h * dis[:, None]
    dis_ref[...] = dis[:, None]


def _s2_body(accp_ref, g1_ref, dis_ref, b1_ref, w2_ref, g2_ref):
    n = g1_ref.shape[0]
    s = accp_ref[0, :n] + accp_ref[1, :n] + g1_ref[...]
    z = jnp.maximum(dis_ref[...] * s + b1_ref[...], 0.0)
    h2 = jnp.dot(z, w2_ref[...], preferred_element_type=jnp.float32)
    g2_ref[...] = dis_ref[...] * h2


def _s3_body(accp_ref, g2_ref, dis_ref, b2_ref, out_ref):
    n = g2_ref.shape[0]
    s = accp_ref[0, :n] + accp_ref[1, :n]
    out_ref[...] = dis_ref[...] * (s[:, None] + g2_ref[...]) + b2_ref[...]


def kernel(x, edge_index, W1, b1, W2, b2):
    n, f = x.shape
    e = edge_index.shape[1]
    src = edge_index[0].astype(jnp.int32)
    dst = edge_index[1].astype(jnp.int32)
    src_p = jnp.concatenate([src, jnp.zeros((EPAD - e,), jnp.int32)])
    dst_p = jnp.concatenate(
        [dst, jnp.full((EPAD - e,), n, jnp.int32)]).reshape(NW * CH, CK)
    zeros2 = jnp.zeros((NPAD, D_HID), jnp.float32)
    zeros1 = jnp.zeros((NPAD,), jnp.float32)
    ones1 = jnp.ones((CH * CK,), jnp.float32)

    deg_p = _scat_const(ones1, src_p, dst_p, zeros1)

    g1, dis = pl.pallas_call(
        _s1_body,
        out_shape=[
            jax.ShapeDtypeStruct((n, D_HID), jnp.float32),
            jax.ShapeDtypeStruct((n, 1), jnp.float32),
        ],
    )(x, W1, deg_p)

    acc1_p = _scat_rows(g1, src_p, dst_p, zeros2)

    g2 = pl.pallas_call(
        _s2_body,
        out_shape=jax.ShapeDtypeStruct((n, 1), jnp.float32),
    )(acc1_p, g1, dis, b1.reshape(1, D_HID), W2)

    acc2_p = _scat_scalar(g2.reshape(n), src_p, dst_p, zeros1)

    out = pl.pallas_call(
        _s3_body,
        out_shape=jax.ShapeDtypeStruct((n, 1), jnp.float32),
    )(acc2_p, g2, dis, b2.reshape(1, 1))

    return out


# speedup vs baseline: 68.2984x; 1.2194x over previous
import functools

import jax
import jax.numpy as jnp
from jax import lax
from jax.experimental import pallas as pl
from jax.experimental.pallas import tpu as pltpu
from jax.experimental.pallas import tpu_sc as plsc

N_NODES = 10000
NPAD = 10240
D_FEAT = 128
D_HID = 16
NC, NS = 2, 16
NW = NC * NS
CK = 128
CH = 80
EPAD = NW * CH * CK
RB = 1024
GRID = NPAD // RB

_mesh = plsc.VectorSubcoreMesh(core_axis_name="c", subcore_axis_name="s")


def _make_edge_scatter(d, gather=True):
    G = 4 if d else 1
    GC = CH // G
    NB = 2 if G > 1 else 1
    EW = CH * CK
    tshape = (NPAD, d) if d else (NPAD,)
    bshape = (NB, GC * CK, d) if d else (NB, GC * CK)
    rps = NPAD // NS

    @functools.partial(
        pl.kernel,
        mesh=_mesh,
        compiler_params=pltpu.CompilerParams(use_tc_tiling_on_sc=False),
        out_type=jax.ShapeDtypeStruct((NC,) + tshape, jnp.float32),
        scratch_types=[
            pltpu.VMEM((EW,), jnp.int32),
            pltpu.VMEM((CH, CK), jnp.int32),
            pltpu.VMEM(bshape, jnp.float32),
            pltpu.VMEM_SHARED(tshape, jnp.float32),
            pltpu.VMEM_SHARED((N_NODES, d) if d else (8,), jnp.float32),
            pltpu.SemaphoreType.DMA,
            pltpu.SemaphoreType.DMA,
        ],
    )
    def scat(table_hbm, src_hbm, dst_hbm, zeros_hbm, out_hbm,
             sidx, didx, rows, acc, tab_sh, gsem, ssem):
        cid = lax.axis_index("c")
        sid = lax.axis_index("s")
        w = cid * NS + sid
        my = pl.ds(sid * rps, rps)
        trs = N_NODES // NS
        pltpu.sync_copy(zeros_hbm.at[my], acc.at[my])
        if gather:
            pltpu.sync_copy(src_hbm.at[pl.ds(w * EW, EW)], sidx)
            pltpu.sync_copy(table_hbm.at[pl.ds(sid * trs, trs)],
                            tab_sh.at[pl.ds(sid * trs, trs)])
        else:
            pltpu.sync_copy(table_hbm, rows.at[0])
        pltpu.sync_copy(dst_hbm.at[pl.ds(w * CH, CH)], didx)
        plsc.subcore_barrier()

        def fire_scatters(buf, g):
            return [pltpu.async_copy(
                        buf.at[pl.ds(j * CK, CK)],
                        acc.at[didx.at[g * GC + j]], ssem, add=True)
                    for j in range(GC)]

        if gather:
            hs = [None] * G
            ss = [None] * G
            hs[0] = pltpu.async_copy(
                tab_sh.at[sidx.at[pl.ds(0, GC * CK)]], rows.at[0], gsem)
            for g in range(G):
                if g + 1 < G:
                    if g >= 1:
                        for h in ss[g - 1]:
                            h.wait()
                    hs[g + 1] = pltpu.async_copy(
                        tab_sh.at[sidx.at[pl.ds((g + 1) * GC * CK, GC * CK)]],
                        rows.at[(g + 1) % NB], gsem)
                hs[g].wait()
                ss[g] = fire_scatters(rows.at[g % NB], g)
            for g in (G - 2, G - 1):
                for h in ss[g]:
                    h.wait()
        else:
            for h in fire_scatters(rows.at[0], 0):
                h.wait()
        plsc.subcore_barrier()
        pltpu.sync_copy(acc.at[my], out_hbm.at[cid].at[my])

    return scat


_scat_rows = _make_edge_scatter(D_HID)
_scat_const = _make_edge_scatter(None, gather=False)

_EW = CH * CK
_GS = 4
_EG = _EW // _GS
_rps = NPAD // NS


@functools.partial(
    pl.kernel,
    mesh=_mesh,
    compiler_params=pltpu.CompilerParams(use_tc_tiling_on_sc=False,
                                         needs_layout_passes=False),
    out_type=jax.ShapeDtypeStruct((NC, NPAD), jnp.float32),
    scratch_types=[
        pltpu.VMEM((N_NODES,), jnp.float32),
        pltpu.VMEM((_EW,), jnp.int32),
        pltpu.VMEM((CH, CK), jnp.int32),
        pltpu.VMEM((_EW,), jnp.float32),
        pltpu.VMEM_SHARED((NPAD,), jnp.float32),
        pltpu.SemaphoreType.DMA,
    ],
)
def _scat_scalar(table_hbm, src_hbm, dst_hbm, zeros_hbm, out_hbm,
                 tabv, sidx, didx, vals, acc, ssem):
    cid = lax.axis_index("c")
    sid = lax.axis_index("s")
    w = cid * NS + sid
    my = pl.ds(sid * _rps, _rps)
    pltpu.sync_copy(zeros_hbm.at[my], acc.at[my])
    pltpu.sync_copy(table_hbm, tabv)
    pltpu.sync_copy(src_hbm.at[pl.ds(w * _EW, _EW)], sidx)
    pltpu.sync_copy(dst_hbm.at[pl.ds(w * CH, CH)], didx)
    plsc.subcore_barrier()

    hs = []
    for g in range(_GS):
        def gbody(i, carry, g=g):
            k = g * _EG + i * 16
            idx = sidx[pl.ds(k, 16)]
            vals[pl.ds(k, 16)] = plsc.load_gather(tabv, [idx])
            return carry

        lax.fori_loop(0, _EG // 16, gbody, 0)
        hs += [pltpu.async_copy(
                   vals.at[pl.ds((g * (CH // _GS) + j) * CK, CK)],
                   acc.at[didx.at[g * (CH // _GS) + j]], ssem, add=True)
               for j in range(CH // _GS)]
    for h in hs:
        h.wait()
    plsc.subcore_barrier()
    pltpu.sync_copy(acc.at[my], out_hbm.at[cid].at[my])


def _s1_body(x_ref, w1_ref, degp_ref, g1_ref, dis_ref):
    n = x_ref.shape[0]
    deg = degp_ref[0, :n] + degp_ref[1, :n] + 1.0
    dis = lax.rsqrt(deg)
    h = jnp.dot(x_ref[...], w1_ref[...], preferred_element_type=jnp.float32)
    g1_ref[...] = h * dis[:, None]
    dis_ref[...] = dis[:, None]


def _s2_body(accp_ref, g1_ref, dis_ref, b1_ref, w2_ref, g2_ref):
    n = g1_ref.shape[0]
    s = accp_ref[0, :n] + accp_ref[1, :n] + g1_ref[...]
    z = jnp.maximum(dis_ref[...] * s + b1_ref[...], 0.0)
    h2 = jnp.dot(z, w2_ref[...], preferred_element_type=jnp.float32)
    g2_ref[...] = dis_ref[...] * h2


def _s3_body(accp_ref, g2_ref, dis_ref, b2_ref, out_ref):
    n = g2_ref.shape[0]
    s = accp_ref[0, :n] + accp_ref[1, :n]
    out_ref[...] = dis_ref[...] * (s[:, None] + g2_ref[...]) + b2_ref[...]


def kernel(x, edge_index, W1, b1, W2, b2):
    n, f = x.shape
    e = edge_index.shape[1]
    src = edge_index[0].astype(jnp.int32)
    dst = edge_index[1].astype(jnp.int32)
    src_p = jnp.concatenate([src, jnp.zeros((EPAD - e,), jnp.int32)])
    dst_p = jnp.concatenate(
        [dst, jnp.full((EPAD - e,), n, jnp.int32)]).reshape(NW * CH, CK)
    zeros2 = jnp.zeros((NPAD, D_HID), jnp.float32)
    zeros1 = jnp.zeros((NPAD,), jnp.float32)
    ones1 = jnp.ones((CH * CK,), jnp.float32)

    deg_p = _scat_const(ones1, src_p, dst_p, zeros1)

    g1, dis = pl.pallas_call(
        _s1_body,
        out_shape=[
            jax.ShapeDtypeStruct((n, D_HID), jnp.float32),
            jax.ShapeDtypeStruct((n, 1), jnp.float32),
        ],
    )(x, W1, deg_p)

    acc1_p = _scat_rows(g1, src_p, dst_p, zeros2)

    g2 = pl.pallas_call(
        _s2_body,
        out_shape=jax.ShapeDtypeStruct((n, 1), jnp.float32),
    )(acc1_p, g1, dis, b1.reshape(1, D_HID), W2)

    acc2_p = _scat_scalar(g2.reshape(n), src_p, dst_p, zeros1)

    out = pl.pallas_call(
        _s3_body,
        out_shape=jax.ShapeDtypeStruct((n, 1), jnp.float32),
    )(acc2_p, g2, dis, b2.reshape(1, 1))

    return out


# speedup vs baseline: 68.9188x; 1.0091x over previous
import functools

import jax
import jax.numpy as jnp
from jax import lax
from jax.experimental import pallas as pl
from jax.experimental.pallas import tpu as pltpu
from jax.experimental.pallas import tpu_sc as plsc

N_NODES = 10000
NPAD = 10240
D_FEAT = 128
D_HID = 16
NC, NS = 2, 16
NW = NC * NS
CK = 128
CH = 80
EPAD = NW * CH * CK
RB = 1024
GRID = NPAD // RB

_mesh = plsc.VectorSubcoreMesh(core_axis_name="c", subcore_axis_name="s")


def _make_edge_scatter(d, gather=True):
    G = 8 if d else 1
    GC = CH // G
    NB = 2 if G > 1 else 1
    EW = CH * CK
    tshape = (NPAD, d) if d else (NPAD,)
    bshape = (NB, GC * CK, d) if d else (NB, GC * CK)
    rps = NPAD // NS

    @functools.partial(
        pl.kernel,
        mesh=_mesh,
        compiler_params=pltpu.CompilerParams(use_tc_tiling_on_sc=False),
        out_type=jax.ShapeDtypeStruct((NC,) + tshape, jnp.float32),
        scratch_types=[
            pltpu.VMEM((EW,), jnp.int32),
            pltpu.VMEM((CH, CK), jnp.int32),
            pltpu.VMEM(bshape, jnp.float32),
            pltpu.VMEM_SHARED(tshape, jnp.float32),
            pltpu.VMEM_SHARED((N_NODES, d) if d else (8,), jnp.float32),
            pltpu.SemaphoreType.DMA,
            pltpu.SemaphoreType.DMA,
        ],
    )
    def scat(table_hbm, src_hbm, dst_hbm, zeros_hbm, out_hbm,
             sidx, didx, rows, acc, tab_sh, gsem, ssem):
        cid = lax.axis_index("c")
        sid = lax.axis_index("s")
        w = cid * NS + sid
        my = pl.ds(sid * rps, rps)
        trs = N_NODES // NS
        pltpu.sync_copy(zeros_hbm.at[my], acc.at[my])
        if gather:
            pltpu.sync_copy(src_hbm.at[pl.ds(w * EW, EW)], sidx)
            pltpu.sync_copy(table_hbm.at[pl.ds(sid * trs, trs)],
                            tab_sh.at[pl.ds(sid * trs, trs)])
        else:
            pltpu.sync_copy(table_hbm, rows.at[0])
        pltpu.sync_copy(dst_hbm.at[pl.ds(w * CH, CH)], didx)
        plsc.subcore_barrier()

        def fire_scatters(buf, g):
            return [pltpu.async_copy(
                        buf.at[pl.ds(j * CK, CK)],
                        acc.at[didx.at[g * GC + j]], ssem, add=True)
                    for j in range(GC)]

        if gather:
            hs = [None] * G
            ss = [None] * G
            hs[0] = pltpu.async_copy(
                tab_sh.at[sidx.at[pl.ds(0, GC * CK)]], rows.at[0], gsem)
            for g in range(G):
                if g + 1 < G:
                    if g >= 1:
                        for h in ss[g - 1]:
                            h.wait()
                    hs[g + 1] = pltpu.async_copy(
                        tab_sh.at[sidx.at[pl.ds((g + 1) * GC * CK, GC * CK)]],
                        rows.at[(g + 1) % NB], gsem)
                hs[g].wait()
                ss[g] = fire_scatters(rows.at[g % NB], g)
            for g in (G - 2, G - 1):
                for h in ss[g]:
                    h.wait()
        else:
            for h in fire_scatters(rows.at[0], 0):
                h.wait()
        plsc.subcore_barrier()
        pltpu.sync_copy(acc.at[my], out_hbm.at[cid].at[my])

    return scat


_scat_rows = _make_edge_scatter(D_HID)
_scat_const = _make_edge_scatter(None, gather=False)

_EW = CH * CK
_GS = 4
_EG = _EW // _GS
_rps = NPAD // NS


@functools.partial(
    pl.kernel,
    mesh=_mesh,
    compiler_params=pltpu.CompilerParams(use_tc_tiling_on_sc=False,
                                         needs_layout_passes=False),
    out_type=jax.ShapeDtypeStruct((NC, NPAD), jnp.float32),
    scratch_types=[
        pltpu.VMEM((N_NODES,), jnp.float32),
        pltpu.VMEM((_EW,), jnp.int32),
        pltpu.VMEM((CH, CK), jnp.int32),
        pltpu.VMEM((_EW,), jnp.float32),
        pltpu.VMEM_SHARED((NPAD,), jnp.float32),
        pltpu.SemaphoreType.DMA,
    ],
)
def _scat_scalar(table_hbm, src_hbm, dst_hbm, zeros_hbm, out_hbm,
                 tabv, sidx, didx, vals, acc, ssem):
    cid = lax.axis_index("c")
    sid = lax.axis_index("s")
    w = cid * NS + sid
    my = pl.ds(sid * _rps, _rps)
    pltpu.sync_copy(zeros_hbm.at[my], acc.at[my])
    pltpu.sync_copy(table_hbm, tabv)
    pltpu.sync_copy(src_hbm.at[pl.ds(w * _EW, _EW)], sidx)
    pltpu.sync_copy(dst_hbm.at[pl.ds(w * CH, CH)], didx)
    plsc.subcore_barrier()

    hs = []
    for g in range(_GS):
        def gbody(i, carry, g=g):
            k = g * _EG + i * 16
            idx = sidx[pl.ds(k, 16)]
            vals[pl.ds(k, 16)] = plsc.load_gather(tabv, [idx])
            return carry

        lax.fori_loop(0, _EG // 16, gbody, 0)
        hs += [pltpu.async_copy(
                   vals.at[pl.ds((g * (CH // _GS) + j) * CK, CK)],
                   acc.at[didx.at[g * (CH // _GS) + j]], ssem, add=True)
               for j in range(CH // _GS)]
    for h in hs:
        h.wait()
    plsc.subcore_barrier()
    pltpu.sync_copy(acc.at[my], out_hbm.at[cid].at[my])


def _s1_body(x_ref, w1_ref, degp_ref, g1_ref, dis_ref):
    n = x_ref.shape[0]
    deg = degp_ref[0, :n] + degp_ref[1, :n] + 1.0
    dis = lax.rsqrt(deg)
    h = jnp.dot(x_ref[...], w1_ref[...], preferred_element_type=jnp.float32)
    g1_ref[...] = h * dis[:, None]
    dis_ref[...] = dis[:, None]


def _s2_body(accp_ref, g1_ref, dis_ref, b1_ref, w2_ref, g2_ref):
    n = g1_ref.shape[0]
    s = accp_ref[0, :n] + accp_ref[1, :n] + g1_ref[...]
    z = jnp.maximum(dis_ref[...] * s + b1_ref[...], 0.0)
    h2 = jnp.dot(z, w2_ref[...], preferred_element_type=jnp.float32)
    g2_ref[...] = dis_ref[...] * h2


def _s3_body(accp_ref, g2_ref, dis_ref, b2_ref, out_ref):
    n = g2_ref.shape[0]
    s = accp_ref[0, :n] + accp_ref[1, :n]
    out_ref[...] = dis_ref[...] * (s[:, None] + g2_ref[...]) + b2_ref[...]


def kernel(x, edge_index, W1, b1, W2, b2):
    n, f = x.shape
    e = edge_index.shape[1]
    src = edge_index[0].astype(jnp.int32)
    dst = edge_index[1].astype(jnp.int32)
    src_p = jnp.concatenate([src, jnp.zeros((EPAD - e,), jnp.int32)])
    dst_p = jnp.concatenate(
        [dst, jnp.full((EPAD - e,), n, jnp.int32)]).reshape(NW * CH, CK)
    zeros2 = jnp.zeros((NPAD, D_HID), jnp.float32)
    zeros1 = jnp.zeros((NPAD,), jnp.float32)
    ones1 = jnp.ones((CH * CK,), jnp.float32)

    deg_p = _scat_const(ones1, src_p, dst_p, zeros1)

    g1, dis = pl.pallas_call(
        _s1_body,
        out_shape=[
            jax.ShapeDtypeStruct((n, D_HID), jnp.float32),
            jax.ShapeDtypeStruct((n, 1), jnp.float32),
        ],
    )(x, W1, deg_p)

    acc1_p = _scat_rows(g1, src_p, dst_p, zeros2)

    g2 = pl.pallas_call(
        _s2_body,
        out_shape=jax.ShapeDtypeStruct((n, 1), jnp.float32),
    )(acc1_p, g1, dis, b1.reshape(1, D_HID), W2)

    acc2_p = _scat_scalar(g2.reshape(n), src_p, dst_p, zeros1)

    out = pl.pallas_call(
        _s3_body,
        out_shape=jax.ShapeDtypeStruct((n, 1), jnp.float32),
    )(acc2_p, g2, dis, b2.reshape(1, 1))

    return out
